# baseline (speedup 1.0000x reference)
import jax
import jax.numpy as jnp
from jax.experimental import pallas as pl
from jax.experimental.pallas import tpu as pltpu

_E = 64
_ROWS = 2048
_HC = 1024


def _fused_body(x_ref, wt_ref, rb_ref, eb_ref, out_ref, sm_ref, ws_ref):
    j = pl.program_id(1)

    @pl.when(j == 0)
    def _router():
        x = x_ref[...]
        logits = jnp.dot(x, wt_ref[...], preferred_element_type=jnp.float32)
        logits = logits + rb_ref[...]
        ml1 = jnp.max(logits, axis=-1, keepdims=True)
        lm = jnp.where(logits == ml1, -jnp.inf, logits)
        ml2 = jnp.max(lm, axis=-1, keepdims=True)
        ex = jnp.exp(logits - ml1)
        r = 1.0 / jnp.sum(ex, axis=-1, keepdims=True)
        sm_ref[...] = (jnp.where(logits >= ml2, ex, 0.0) * r).astype(
            jnp.bfloat16)
        ws_ref[...] = (1.0 + jnp.exp(ml2 - ml1)) * r

    xh = x_ref[:, pl.ds(j * _HC, _HC)]
    comb = jnp.dot(sm_ref[...], eb_ref[...],
                   preferred_element_type=jnp.float32)
    out_ref[...] = ws_ref[...] * xh + comb


def kernel(hidden_states, router_weight, router_bias, expert_bias):
    B, S, H = hidden_states.shape
    BS = B * S
    flat = hidden_states.reshape(BS, H)
    wt = router_weight.T
    rb = router_bias.reshape(1, _E)
    eb16 = expert_bias.astype(jnp.bfloat16)

    out = pl.pallas_call(
        _fused_body,
        grid=(BS // _ROWS, H // _HC),
        in_specs=[
            pl.BlockSpec((_ROWS, H), lambda i, j: (i, 0)),
            pl.BlockSpec((H, _E), lambda i, j: (0, 0)),
            pl.BlockSpec((1, _E), lambda i, j: (0, 0)),
            pl.BlockSpec((_E, _HC), lambda i, j: (0, j)),
        ],
        out_specs=pl.BlockSpec((_ROWS, _HC), lambda i, j: (i, j)),
        out_shape=jax.ShapeDtypeStruct((BS, H), jnp.float32),
        scratch_shapes=[
            pltpu.VMEM((_ROWS, _E), jnp.bfloat16),
            pltpu.VMEM((_ROWS, 1), jnp.float32),
        ],
        compiler_params=pltpu.CompilerParams(
            vmem_limit_bytes=100 * 1024 * 1024),
    )(flat, wt, rb, eb16)
    return out.reshape(B, S, H)


# speedup vs baseline: 1.2441x; 1.2441x over previous
import jax
import jax.numpy as jnp
from jax.experimental import pallas as pl
from jax.experimental.pallas import tpu as pltpu

_E = 64
_ROWS = 1024


def _fused_body(x_ref, wt_ref, rb_ref, eb_ref, out_ref):
    x = x_ref[...]
    logits = jnp.dot(x, wt_ref[...], preferred_element_type=jnp.float32)
    logits = logits + rb_ref[...]

    ml1 = jnp.max(logits, axis=-1, keepdims=True)
    lm = jnp.where(logits == ml1, -jnp.inf, logits)
    ml2 = jnp.max(lm, axis=-1, keepdims=True)

    ex = jnp.exp(logits - ml1)
    r = 1.0 / jnp.sum(ex, axis=-1, keepdims=True)

    s_masked = jnp.where(logits >= ml2, ex, 0.0) * r
    wsum = (1.0 + jnp.exp(ml2 - ml1)) * r

    comb = jnp.dot(s_masked.astype(jnp.bfloat16), eb_ref[...],
                   preferred_element_type=jnp.float32)
    out_ref[...] = wsum * x + comb


def kernel(hidden_states, router_weight, router_bias, expert_bias):
    B, S, H = hidden_states.shape
    BS = B * S
    flat = hidden_states.reshape(BS, H)
    wt = router_weight.T
    rb = router_bias.reshape(1, _E)
    eb16 = expert_bias.astype(jnp.bfloat16)

    out = pl.pallas_call(
        _fused_body,
        grid=(BS // _ROWS,),
        in_specs=[
            pl.BlockSpec((_ROWS, H), lambda i: (i, 0)),
            pl.BlockSpec((H, _E), lambda i: (0, 0)),
            pl.BlockSpec((1, _E), lambda i: (0, 0)),
            pl.BlockSpec((_E, H), lambda i: (0, 0)),
        ],
        out_specs=pl.BlockSpec((_ROWS, H), lambda i: (i, 0)),
        out_shape=jax.ShapeDtypeStruct((BS, H), jnp.float32),
        compiler_params=pltpu.CompilerParams(
            dimension_semantics=("parallel",)),
    )(flat, wt, rb, eb16)
    return out.reshape(B, S, H)
